# Initial kernel scaffold; baseline (speedup 1.0000x reference)
import jax
import jax.numpy as jnp
from jax.experimental import pallas as pl


def kernel(x, edge_index, W1, att1, b1, W2, b2):
    raise NotImplementedError("write your pallas kernel here")


# speedup vs baseline: 1.0005x; 1.0005x over previous
import functools

import jax
import jax.numpy as jnp
from jax import lax
from jax.experimental import pallas as pl
from jax.experimental.pallas import tpu as pltpu

_N = 10000
_E = 320000
_D = 128
_H = 8
_F = 8
_HF = _H * _F
_O2 = 8


def _mm_kernel(x_ref, w_ref, o_ref):
    o_ref[...] = jnp.dot(x_ref[...], w_ref[...],
                         preferred_element_type=jnp.float32)


def _matmul(x, w, block_rows):
    n, k = x.shape
    _, m = w.shape
    grid = n // block_rows
    return pl.pallas_call(
        _mm_kernel,
        grid=(grid,),
        in_specs=[
            pl.BlockSpec((block_rows, k), lambda i: (i, 0)),
            pl.BlockSpec((k, m), lambda i: (0, 0)),
        ],
        out_specs=pl.BlockSpec((block_rows, m), lambda i: (i, 0)),
        out_shape=jax.ShapeDtypeStruct((n, m), jnp.float32),
    )(x, w)


def _seg_softmax(logits, seg, num_segments):
    m = jax.ops.segment_max(logits, seg, num_segments=num_segments)
    m = jnp.where(jnp.isfinite(m), m, 0.0)
    e = jnp.exp(logits - m[seg])
    s = jax.ops.segment_sum(e, seg, num_segments=num_segments)
    return e / (s[seg] + 1e-16)


def _hconv(xw, idx_node, idx_edge, alpha, num_nodes, num_edges):
    ones = jnp.ones((idx_edge.shape[0],), dtype=xw.dtype)
    deg = jax.ops.segment_sum(ones, idx_node, num_segments=num_nodes)
    Dinv = jnp.where(deg > 0, 1.0 / deg, 0.0)
    bsum = jax.ops.segment_sum(ones, idx_edge, num_segments=num_edges)
    Binv = jnp.where(bsum > 0, 1.0 / bsum, 0.0)
    msg1 = xw[idx_node]
    if alpha is not None:
        msg1 = msg1 * alpha[:, :, None]
    msg1 = msg1 * Binv[idx_edge][:, None, None]
    edge_feat = jax.ops.segment_sum(msg1, idx_edge, num_segments=num_edges)
    msg2 = edge_feat[idx_edge]
    if alpha is not None:
        msg2 = msg2 * alpha[:, :, None]
    msg2 = msg2 * Dinv[idx_node][:, None, None]
    return jax.ops.segment_sum(msg2, idx_node, num_segments=num_nodes)


def kernel(x, edge_index, W1, att1, b1, W2, b2):
    idx_node = edge_index[0]
    idx_edge = edge_index[1]
    xw = _matmul(x, W1, 1000).reshape(-1, _H, _F)
    x_i = xw[idx_node]
    x_j = xw[idx_edge]
    alpha = (jnp.concatenate([x_i, x_j], axis=-1) * att1).sum(axis=-1)
    alpha = jax.nn.leaky_relu(alpha, negative_slope=0.2)
    alpha = _seg_softmax(alpha, idx_node, _N)
    h = _hconv(xw, idx_node, idx_edge, alpha, _N, _N)
    h = h.reshape(_N, _HF) + b1
    h = jax.nn.elu(h)
    hw = _matmul(h, W2, 1000)[:, None, :]
    o = _hconv(hw, idx_node, idx_edge, None, _N, _N)
    o = o.reshape(_N, _O2) + b2
    return jax.nn.log_softmax(o, axis=1)


# speedup vs baseline: 61.7105x; 61.6770x over previous
import functools

import jax
import jax.numpy as jnp
from jax import lax
from jax.experimental import pallas as pl
from jax.experimental.pallas import tpu as pltpu
from jax.experimental.pallas import tpu_sc as plsc

_N = 10000
_E = 320000
_D = 128
_H = 8
_F = 8
_HF = _H * _F
_O2 = 8

_NC = 2
_NS = 16
_NW = _NC * _NS
_PER = _E // _NW
_CH = 128
_NCH = _PER // _CH
_REM = _PER - _NCH * _CH
_NPAD = 10240
_STRIPE = _NPAD // _NS

_SC_MESH = plsc.VectorSubcoreMesh(core_axis_name="c", subcore_axis_name="s")


def _p1_body(u_hbm, v_hbm, t1_hbm, t2_hbm, c_hbm,
             e2_hbm, pa_hbm, pb_hbm,
             uref, vref, uref2, vref2, tu, tv, tu2, tv2,
             src, src2, zbuf, cvm, accA, accB):
    c = lax.axis_index("c")
    s = lax.axis_index("s")
    wid = s * _NC + c
    base = wid * _PER

    zero = jnp.zeros((16,), jnp.float32)

    def zrow(i, _):
        zbuf[i, :] = zero
        return 0

    lax.fori_loop(0, _STRIPE, zrow, 0)
    pltpu.sync_copy(zbuf, accA.at[pl.ds(s * _STRIPE, _STRIPE)])
    pltpu.sync_copy(zbuf, accB.at[pl.ds(s * _STRIPE, _STRIPE)])
    pltpu.sync_copy(c_hbm, cvm)
    plsc.subcore_barrier()

    cvec = cvm[...]
    lane = lax.iota(jnp.int32, 16)
    low = lane < 8
    pat = jnp.where(lane == 8, 1.0, 0.0).astype(jnp.float32)

    def compute(n, tu_r, tv_r, src_r):
        def body(i, _):
            l = tu_r[i, :] + tv_r[i, :]
            l = jnp.where(l >= 0, l, 0.2 * l)
            e = jnp.exp(l - cvec)
            src_r[i, :] = jnp.where(low, e, pat)
            return 0

        lax.fori_loop(0, n, body, 0)

    def chunk(j, _):
        off = base + j * _CH
        pltpu.sync_copy(u_hbm.at[pl.ds(off, _CH)], uref.at[0])
        pltpu.sync_copy(v_hbm.at[pl.ds(off, _CH)], vref.at[0])
        pltpu.sync_copy(t1_hbm.at[uref.at[0]], tu)
        pltpu.sync_copy(t2_hbm.at[vref.at[0]], tv)
        compute(_CH, tu, tv, src)
        pltpu.sync_copy(src, e2_hbm.at[pl.ds(off, _CH)])
        pltpu.sync_copy(src, accA.at[uref.at[0]], add=True)
        pltpu.sync_copy(src, accB.at[vref.at[0]], add=True)
        return 0

    lax.fori_loop(0, _NCH, chunk, 0)

    off = base + _NCH * _CH
    pltpu.sync_copy(u_hbm.at[pl.ds(off, _REM)], uref2.at[0])
    pltpu.sync_copy(v_hbm.at[pl.ds(off, _REM)], vref2.at[0])
    pltpu.sync_copy(t1_hbm.at[uref2.at[0]], tu2)
    pltpu.sync_copy(t2_hbm.at[vref2.at[0]], tv2)
    compute(_REM, tu2, tv2, src2)
    pltpu.sync_copy(src2, e2_hbm.at[pl.ds(off, _REM)])
    pltpu.sync_copy(src2, accA.at[uref2.at[0]], add=True)
    pltpu.sync_copy(src2, accB.at[vref2.at[0]], add=True)

    plsc.subcore_barrier()
    pltpu.sync_copy(accA.at[pl.ds(s * _STRIPE, _STRIPE)],
                    pa_hbm.at[c, pl.ds(s * _STRIPE, _STRIPE)])
    pltpu.sync_copy(accB.at[pl.ds(s * _STRIPE, _STRIPE)],
                    pb_hbm.at[c, pl.ds(s * _STRIPE, _STRIPE)])


def _p1(u, v, t1, t2, c16, interpret=False):
    return pl.kernel(
        _p1_body,
        out_type=[
            jax.ShapeDtypeStruct((_E, 16), jnp.float32),
            jax.ShapeDtypeStruct((_NC, _NPAD, 16), jnp.float32),
            jax.ShapeDtypeStruct((_NC, _NPAD, 16), jnp.float32),
        ],
        mesh=_SC_MESH,
        scratch_types=[
            pltpu.VMEM((1, _CH), jnp.int32),
            pltpu.VMEM((1, _CH), jnp.int32),
            pltpu.VMEM((1, _REM), jnp.int32),
            pltpu.VMEM((1, _REM), jnp.int32),
            pltpu.VMEM((_CH, 16), jnp.float32),
            pltpu.VMEM((_CH, 16), jnp.float32),
            pltpu.VMEM((_REM, 16), jnp.float32),
            pltpu.VMEM((_REM, 16), jnp.float32),
            pltpu.VMEM((_CH, 16), jnp.float32),
            pltpu.VMEM((_REM, 16), jnp.float32),
            pltpu.VMEM((_STRIPE, 16), jnp.float32),
            pltpu.VMEM((16,), jnp.float32),
            pltpu.VMEM_SHARED((_NPAD, 16), jnp.float32),
            pltpu.VMEM_SHARED((_NPAD, 16), jnp.float32),
        ],
        compiler_params=pltpu.CompilerParams(use_tc_tiling_on_sc=False),
        interpret=interpret,
    )(u, v, t1, t2, c16)


def _make_edge_pass(W, with_alpha):
    NQ = W // 16

    def body(*refs):
        if with_alpha:
            (gi_hbm, si_hbm, t_hbm, e2_hbm, out_hbm,
             giref, siref, giref2, siref2,
             tbuf, tbuf2, ebuf, ebuf2, msg, msg2, zbuf, acc) = refs
        else:
            (gi_hbm, si_hbm, t_hbm, out_hbm,
             giref, siref, giref2, siref2,
             tbuf, tbuf2, zbuf, acc) = refs
        c = lax.axis_index("c")
        s = lax.axis_index("s")
        wid = s * _NC + c
        base = wid * _PER
        zero = jnp.zeros((16,), jnp.float32)

        def zrow(i, _):
            for q in range(NQ):
                zbuf[i, pl.ds(16 * q, 16)] = zero
            return 0

        lax.fori_loop(0, _STRIPE, zrow, 0)
        pltpu.sync_copy(zbuf, acc.at[pl.ds(s * _STRIPE, _STRIPE)])
        plsc.subcore_barrier()

        if with_alpha:
            lane = lax.iota(jnp.int32, 16)
            low = lane < 8

        def do_chunk(off, n, giref_, siref_, tbuf_, ebuf_, msg_):
            pltpu.sync_copy(gi_hbm.at[pl.ds(off, n)], giref_.at[0])
            pltpu.sync_copy(si_hbm.at[pl.ds(off, n)], siref_.at[0])
            pltpu.sync_copy(t_hbm.at[giref_.at[0]], tbuf_)
            if with_alpha:
                pltpu.sync_copy(e2_hbm.at[pl.ds(off, n)], ebuf_)

                def per_edge(e, _):
                    ev = ebuf_[e, :]
                    for q in range(NQ):
                        a0 = jnp.full((16,), ev[2 * q], jnp.float32)
                        a1 = jnp.full((16,), ev[2 * q + 1], jnp.float32)
                        a = jnp.where(low, a0, a1)
                        msg_[e, pl.ds(16 * q, 16)] = (
                            tbuf_[e, pl.ds(16 * q, 16)] * a)
                    return 0

                lax.fori_loop(0, n, per_edge, 0)
                pltpu.sync_copy(msg_, acc.at[siref_.at[0]], add=True)
            else:
                pltpu.sync_copy(tbuf_, acc.at[siref_.at[0]], add=True)

        def chunk(j, _):
            do_chunk(base + j * _CH, _CH, giref, siref, tbuf, ebuf, msg)
            return 0

        if with_alpha:
            lax.fori_loop(0, _NCH, chunk, 0)
            do_chunk(base + _NCH * _CH, _REM, giref2, siref2, tbuf2, ebuf2,
                     msg2)
        else:
            def chunk0(j, _):
                do_chunk(base + j * _CH, _CH, giref, siref, tbuf, None, None)
                return 0

            lax.fori_loop(0, _NCH, chunk0, 0)
            do_chunk(base + _NCH * _CH, _REM, giref2, siref2, tbuf2, None,
                     None)

        plsc.subcore_barrier()
        pltpu.sync_copy(acc.at[pl.ds(s * _STRIPE, _STRIPE)],
                        out_hbm.at[c, pl.ds(s * _STRIPE, _STRIPE)])

    return body


_P23_BODY = _make_edge_pass(64, True)
_P45_BODY = _make_edge_pass(16, False)


def _p23(gi, si, table, e2):
    return pl.kernel(
        _P23_BODY,
        out_type=jax.ShapeDtypeStruct((_NC, _NPAD, 64), jnp.float32),
        mesh=_SC_MESH,
        scratch_types=[
            pltpu.VMEM((1, _CH), jnp.int32),
            pltpu.VMEM((1, _CH), jnp.int32),
            pltpu.VMEM((1, _REM), jnp.int32),
            pltpu.VMEM((1, _REM), jnp.int32),
            pltpu.VMEM((_CH, 64), jnp.float32),
            pltpu.VMEM((_REM, 64), jnp.float32),
            pltpu.VMEM((_CH, 16), jnp.float32),
            pltpu.VMEM((_REM, 16), jnp.float32),
            pltpu.VMEM((_CH, 64), jnp.float32),
            pltpu.VMEM((_REM, 64), jnp.float32),
            pltpu.VMEM((_STRIPE, 64), jnp.float32),
            pltpu.VMEM_SHARED((_NPAD, 64), jnp.float32),
        ],
        compiler_params=pltpu.CompilerParams(use_tc_tiling_on_sc=False),
    )(gi, si, table, e2)


def _p45(gi, si, table):
    return pl.kernel(
        _P45_BODY,
        out_type=jax.ShapeDtypeStruct((_NC, _NPAD, 16), jnp.float32),
        mesh=_SC_MESH,
        scratch_types=[
            pltpu.VMEM((1, _CH), jnp.int32),
            pltpu.VMEM((1, _CH), jnp.int32),
            pltpu.VMEM((1, _REM), jnp.int32),
            pltpu.VMEM((1, _REM), jnp.int32),
            pltpu.VMEM((_CH, 16), jnp.float32),
            pltpu.VMEM((_REM, 16), jnp.float32),
            pltpu.VMEM((_STRIPE, 16), jnp.float32),
            pltpu.VMEM_SHARED((_NPAD, 16), jnp.float32),
        ],
        compiler_params=pltpu.CompilerParams(use_tc_tiling_on_sc=False),
    )(gi, si, table)


def _mm_kernel(x_ref, w_ref, o_ref):
    o_ref[...] = jnp.dot(x_ref[...], w_ref[...],
                         preferred_element_type=jnp.float32)


def _matmul(x, w, block_rows):
    n, k = x.shape
    _, m = w.shape
    grid = n // block_rows
    return pl.pallas_call(
        _mm_kernel,
        grid=(grid,),
        in_specs=[
            pl.BlockSpec((block_rows, k), lambda i: (i, 0)),
            pl.BlockSpec((k, m), lambda i: (0, 0)),
        ],
        out_specs=pl.BlockSpec((block_rows, m), lambda i: (i, 0)),
        out_shape=jax.ShapeDtypeStruct((n, m), jnp.float32),
    )(x, w)


def _segsum(vals, seg, num_segments):
    return jax.ops.segment_sum(vals, seg, num_segments=num_segments)


def kernel(x, edge_index, W1, att1, b1, W2, b2, interpret=False):
    idx_node = edge_index[0]
    idx_edge = edge_index[1]
    xw = _matmul(x, W1, 1000)
    xw3 = xw.reshape(_N, _H, _F)
    a_src = (xw3 * att1[:, :, :_F]).sum(-1)
    a_dst = (xw3 * att1[:, :, _F:]).sum(-1)
    c = jax.nn.leaky_relu(a_src.max(0) + a_dst.max(0), negative_slope=0.2)
    c16 = jnp.concatenate([c, jnp.zeros((8,), jnp.float32)])
    t1 = jnp.concatenate([a_src, a_dst], axis=1)
    t2 = jnp.concatenate([a_dst, a_src], axis=1)

    e2, pa, pb = _p1(idx_node, idx_edge, t1, t2, c16, interpret=interpret)
    A = pa[0, :_N] + pa[1, :_N]
    B = pb[0, :_N] + pb[1, :_N]
    s = A[:, :_H]
    deg = A[:, _H]
    bsum = B[:, _H]
    sinv = 1.0 / (s + 1e-16)
    Dinv = jnp.where(deg > 0, 1.0 / deg, 0.0)
    Binv = jnp.where(bsum > 0, 1.0 / bsum, 0.0)

    xs = (xw3 * sinv[:, :, None]).reshape(_N, _HF)
    efp = _p23(idx_node, idx_edge, xs, e2)
    ef = (efp[0, :_N] + efp[1, :_N]) * Binv[:, None]

    hp = _p23(idx_edge, idx_node, ef, e2)
    h = (hp[0, :_N] + hp[1, :_N]).reshape(_N, _H, _F)
    h = h * (sinv * Dinv[:, None])[:, :, None]
    h = jax.nn.elu(h.reshape(_N, _HF) + b1)

    hw = _matmul(h, W2, 1000)
    pad8 = jnp.zeros((_N, 8), jnp.float32)
    hw16 = jnp.concatenate([hw, pad8], axis=1)

    e2p = _p45(idx_node, idx_edge, hw16)
    ef2 = (e2p[0, :_N, :_O2] + e2p[1, :_N, :_O2]) * Binv[:, None]
    ef2_16 = jnp.concatenate([ef2, pad8], axis=1)

    op = _p45(idx_edge, idx_node, ef2_16)
    o = (op[0, :_N, :_O2] + op[1, :_N, :_O2]) * Dinv[:, None]
    o = o + b2
    return jax.nn.log_softmax(o, axis=1)


# speedup vs baseline: 92.1439x; 1.4932x over previous
import functools

import jax
import jax.numpy as jnp
from jax import lax
from jax.experimental import pallas as pl
from jax.experimental.pallas import tpu as pltpu
from jax.experimental.pallas import tpu_sc as plsc

_N = 10000
_E = 320000
_D = 128
_H = 8
_F = 8
_HF = _H * _F
_O2 = 8

_NC = 2
_NS = 16
_NW = _NC * _NS
_PER = _E // _NW
_CH = 128
_NCH = _PER // _CH
_REM = _PER - _NCH * _CH
_NPAD = 10240
_STRIPE = _NPAD // _NS

_SC_MESH = plsc.VectorSubcoreMesh(core_axis_name="c", subcore_axis_name="s")


def _p1_body(u_hbm, v_hbm, t1_hbm, t2_hbm, c_hbm,
             e2_hbm, pa_hbm, pb_hbm,
             u0, u1, v0, v1, u2, v2,
             tu0, tu1, tu2, tv0, tv1, tv2,
             s0, s1, s2, zbuf, cvm, accA, accB, sem0, sem1):
    c = lax.axis_index("c")
    s = lax.axis_index("s")
    wid = s * _NC + c
    base = wid * _PER

    zero = jnp.zeros((16,), jnp.float32)

    def zrow(i, _):
        zbuf[i, :] = zero
        return 0

    lax.fori_loop(0, _STRIPE, zrow, 0)
    pltpu.sync_copy(zbuf, accA.at[pl.ds(s * _STRIPE, _STRIPE)])
    pltpu.sync_copy(zbuf, accB.at[pl.ds(s * _STRIPE, _STRIPE)])
    pltpu.sync_copy(c_hbm, cvm)
    plsc.subcore_barrier()

    cvec = cvm[...]
    lane = lax.iota(jnp.int32, 16)
    low = lane < 8
    pat = jnp.where(lane == 8, 1.0, 0.0).astype(jnp.float32)

    def compute(n, tu_r, tv_r, src_r):
        def body(i, _):
            l = tu_r[i, :] + tv_r[i, :]
            l = jnp.where(l >= 0, l, 0.2 * l)
            e = jnp.exp(l - cvec)
            src_r[i, :] = jnp.where(low, e, pat)
            return 0

        lax.fori_loop(0, n, body, 0)

    def load_idx(j, ur, vr):
        off = base + j * _CH
        pltpu.sync_copy(u_hbm.at[pl.ds(off, _CH)], ur.at[0])
        pltpu.sync_copy(v_hbm.at[pl.ds(off, _CH)], vr.at[0])

    def start_g(ur, vr, tur, tvr, sem):
        pltpu.async_copy(t1_hbm.at[ur.at[0]], tur, sem)
        pltpu.async_copy(t2_hbm.at[vr.at[0]], tvr, sem)

    def finish(j, ur, vr, tur, tvr, sr, sem):
        pltpu.make_async_copy(t1_hbm.at[ur.at[0]], tur, sem).wait()
        pltpu.make_async_copy(t2_hbm.at[vr.at[0]], tvr, sem).wait()
        compute(_CH, tur, tvr, sr)
        off = base + j * _CH
        pltpu.sync_copy(sr, e2_hbm.at[pl.ds(off, _CH)])
        pltpu.sync_copy(sr, accA.at[ur.at[0]], add=True)
        pltpu.sync_copy(sr, accB.at[vr.at[0]], add=True)

    load_idx(0, u0, v0)
    start_g(u0, v0, tu0, tv0, sem0)

    NPAIR = _NCH // 2

    def pair(jj, _):
        j0 = 2 * jj
        j1 = j0 + 1
        load_idx(j1, u1, v1)
        start_g(u1, v1, tu1, tv1, sem1)
        finish(j0, u0, v0, tu0, tv0, s0, sem0)

        @pl.when(jj < NPAIR - 1)
        def _():
            load_idx(j0 + 2, u0, v0)
            start_g(u0, v0, tu0, tv0, sem0)

        finish(j1, u1, v1, tu1, tv1, s1, sem1)
        return 0

    lax.fori_loop(0, NPAIR, pair, 0)

    off = base + _NCH * _CH
    pltpu.sync_copy(u_hbm.at[pl.ds(off, _REM)], u2.at[0])
    pltpu.sync_copy(v_hbm.at[pl.ds(off, _REM)], v2.at[0])
    pltpu.sync_copy(t1_hbm.at[u2.at[0]], tu2)
    pltpu.sync_copy(t2_hbm.at[v2.at[0]], tv2)
    compute(_REM, tu2, tv2, s2)
    pltpu.sync_copy(s2, e2_hbm.at[pl.ds(off, _REM)])
    pltpu.sync_copy(s2, accA.at[u2.at[0]], add=True)
    pltpu.sync_copy(s2, accB.at[v2.at[0]], add=True)

    plsc.subcore_barrier()
    pltpu.sync_copy(accA.at[pl.ds(s * _STRIPE, _STRIPE)],
                    pa_hbm.at[c, pl.ds(s * _STRIPE, _STRIPE)])
    pltpu.sync_copy(accB.at[pl.ds(s * _STRIPE, _STRIPE)],
                    pb_hbm.at[c, pl.ds(s * _STRIPE, _STRIPE)])


def _p1(u, v, t1, t2, c16, interpret=False):
    return pl.kernel(
        _p1_body,
        out_type=[
            jax.ShapeDtypeStruct((_E, 16), jnp.float32),
            jax.ShapeDtypeStruct((_NC, _NPAD, 16), jnp.float32),
            jax.ShapeDtypeStruct((_NC, _NPAD, 16), jnp.float32),
        ],
        mesh=_SC_MESH,
        scratch_types=[
            pltpu.VMEM((1, _CH), jnp.int32),
            pltpu.VMEM((1, _CH), jnp.int32),
            pltpu.VMEM((1, _CH), jnp.int32),
            pltpu.VMEM((1, _CH), jnp.int32),
            pltpu.VMEM((1, _REM), jnp.int32),
            pltpu.VMEM((1, _REM), jnp.int32),
            pltpu.VMEM((_CH, 16), jnp.float32),
            pltpu.VMEM((_CH, 16), jnp.float32),
            pltpu.VMEM((_REM, 16), jnp.float32),
            pltpu.VMEM((_CH, 16), jnp.float32),
            pltpu.VMEM((_CH, 16), jnp.float32),
            pltpu.VMEM((_REM, 16), jnp.float32),
            pltpu.VMEM((_CH, 16), jnp.float32),
            pltpu.VMEM((_CH, 16), jnp.float32),
            pltpu.VMEM((_REM, 16), jnp.float32),
            pltpu.VMEM((_STRIPE, 16), jnp.float32),
            pltpu.VMEM((16,), jnp.float32),
            pltpu.VMEM_SHARED((_NPAD, 16), jnp.float32),
            pltpu.VMEM_SHARED((_NPAD, 16), jnp.float32),
            pltpu.SemaphoreType.DMA,
            pltpu.SemaphoreType.DMA,
        ],
        compiler_params=pltpu.CompilerParams(use_tc_tiling_on_sc=False),
        interpret=interpret,
    )(u, v, t1, t2, c16)


def _make_edge_pass(W, with_alpha):
    NQ = W // 16
    NPAIR = _NCH // 2

    def body(*refs):
        if with_alpha:
            (gi_hbm, si_hbm, t_hbm, e2_hbm, out_hbm,
             gi0, gi1, si0, si1, gi2, si2,
             tb0, tb1, tb2, eb0, eb1, eb2, ms0, ms1, ms2,
             zbuf, acc, sem0, sem1) = refs
        else:
            (gi_hbm, si_hbm, t_hbm, out_hbm,
             gi0, gi1, si0, si1, gi2, si2,
             tb0, tb1, tb2,
             zbuf, acc, sem0, sem1) = refs
        c = lax.axis_index("c")
        s = lax.axis_index("s")
        wid = s * _NC + c
        base = wid * _PER
        zero = jnp.zeros((16,), jnp.float32)

        def zrow(i, _):
            for q in range(NQ):
                zbuf[i, pl.ds(16 * q, 16)] = zero
            return 0

        lax.fori_loop(0, _STRIPE, zrow, 0)
        pltpu.sync_copy(zbuf, acc.at[pl.ds(s * _STRIPE, _STRIPE)])
        plsc.subcore_barrier()

        lane = lax.iota(jnp.int32, 16)
        low = lane < 8

        def load_idx(j, gir, sir):
            off = base + j * _CH
            pltpu.sync_copy(gi_hbm.at[pl.ds(off, _CH)], gir.at[0])
            pltpu.sync_copy(si_hbm.at[pl.ds(off, _CH)], sir.at[0])

        def start_g(j, gir, tbr, ebr, sem):
            pltpu.async_copy(t_hbm.at[gir.at[0]], tbr, sem)
            if with_alpha:
                off = base + j * _CH
                pltpu.async_copy(e2_hbm.at[pl.ds(off, _CH)], ebr, sem)

        def wait_g(j, gir, tbr, ebr, sem):
            pltpu.make_async_copy(t_hbm.at[gir.at[0]], tbr, sem).wait()
            if with_alpha:
                off = base + j * _CH
                pltpu.make_async_copy(
                    e2_hbm.at[pl.ds(off, _CH)], ebr, sem).wait()

        def compute(n, tbr, ebr, msr):
            def per_edge(e, _):
                ev = ebr[e, :]
                for q in range(NQ):
                    a0 = jnp.full((16,), ev[2 * q], jnp.float32)
                    a1 = jnp.full((16,), ev[2 * q + 1], jnp.float32)
                    a = jnp.where(low, a0, a1)
                    msr[e, pl.ds(16 * q, 16)] = (
                        tbr[e, pl.ds(16 * q, 16)] * a)
                return 0

            lax.fori_loop(0, n, per_edge, 0)

        def finish(j, gir, sir, tbr, ebr, msr, sem):
            wait_g(j, gir, tbr, ebr, sem)
            if with_alpha:
                compute(_CH, tbr, ebr, msr)
                pltpu.sync_copy(msr, acc.at[sir.at[0]], add=True)
            else:
                pltpu.sync_copy(tbr, acc.at[sir.at[0]], add=True)

        eb0_, eb1_, ms0_, ms1_ = ((eb0, eb1, ms0, ms1) if with_alpha
                                  else (None, None, None, None))

        load_idx(0, gi0, si0)
        start_g(0, gi0, tb0, eb0_, sem0)

        def pair(jj, _):
            j0 = 2 * jj
            j1 = j0 + 1
            load_idx(j1, gi1, si1)
            start_g(j1, gi1, tb1, eb1_, sem1)
            finish(j0, gi0, si0, tb0, eb0_, ms0_, sem0)

            @pl.when(jj < NPAIR - 1)
            def _():
                load_idx(j0 + 2, gi0, si0)
                start_g(j0 + 2, gi0, tb0, eb0_, sem0)

            finish(j1, gi1, si1, tb1, eb1_, ms1_, sem1)
            return 0

        lax.fori_loop(0, NPAIR, pair, 0)

        off = base + _NCH * _CH
        pltpu.sync_copy(gi_hbm.at[pl.ds(off, _REM)], gi2.at[0])
        pltpu.sync_copy(si_hbm.at[pl.ds(off, _REM)], si2.at[0])
        pltpu.sync_copy(t_hbm.at[gi2.at[0]], tb2)
        if with_alpha:
            pltpu.sync_copy(e2_hbm.at[pl.ds(off, _REM)], eb2)
            compute(_REM, tb2, eb2, ms2)
            pltpu.sync_copy(ms2, acc.at[si2.at[0]], add=True)
        else:
            pltpu.sync_copy(tb2, acc.at[si2.at[0]], add=True)

        plsc.subcore_barrier()
        pltpu.sync_copy(acc.at[pl.ds(s * _STRIPE, _STRIPE)],
                        out_hbm.at[c, pl.ds(s * _STRIPE, _STRIPE)])

    return body


_P23_BODY = _make_edge_pass(64, True)
_P45_BODY = _make_edge_pass(16, False)


def _p23(gi, si, table, e2):
    return pl.kernel(
        _P23_BODY,
        out_type=jax.ShapeDtypeStruct((_NC, _NPAD, 64), jnp.float32),
        mesh=_SC_MESH,
        scratch_types=[
            pltpu.VMEM((1, _CH), jnp.int32),
            pltpu.VMEM((1, _CH), jnp.int32),
            pltpu.VMEM((1, _CH), jnp.int32),
            pltpu.VMEM((1, _CH), jnp.int32),
            pltpu.VMEM((1, _REM), jnp.int32),
            pltpu.VMEM((1, _REM), jnp.int32),
            pltpu.VMEM((_CH, 64), jnp.float32),
            pltpu.VMEM((_CH, 64), jnp.float32),
            pltpu.VMEM((_REM, 64), jnp.float32),
            pltpu.VMEM((_CH, 16), jnp.float32),
            pltpu.VMEM((_CH, 16), jnp.float32),
            pltpu.VMEM((_REM, 16), jnp.float32),
            pltpu.VMEM((_CH, 64), jnp.float32),
            pltpu.VMEM((_CH, 64), jnp.float32),
            pltpu.VMEM((_REM, 64), jnp.float32),
            pltpu.VMEM((_STRIPE, 64), jnp.float32),
            pltpu.VMEM_SHARED((_NPAD, 64), jnp.float32),
            pltpu.SemaphoreType.DMA,
            pltpu.SemaphoreType.DMA,
        ],
        compiler_params=pltpu.CompilerParams(use_tc_tiling_on_sc=False),
    )(gi, si, table, e2)


def _p45(gi, si, table):
    return pl.kernel(
        _P45_BODY,
        out_type=jax.ShapeDtypeStruct((_NC, _NPAD, 16), jnp.float32),
        mesh=_SC_MESH,
        scratch_types=[
            pltpu.VMEM((1, _CH), jnp.int32),
            pltpu.VMEM((1, _CH), jnp.int32),
            pltpu.VMEM((1, _CH), jnp.int32),
            pltpu.VMEM((1, _CH), jnp.int32),
            pltpu.VMEM((1, _REM), jnp.int32),
            pltpu.VMEM((1, _REM), jnp.int32),
            pltpu.VMEM((_CH, 16), jnp.float32),
            pltpu.VMEM((_CH, 16), jnp.float32),
            pltpu.VMEM((_REM, 16), jnp.float32),
            pltpu.VMEM((_STRIPE, 16), jnp.float32),
            pltpu.VMEM_SHARED((_NPAD, 16), jnp.float32),
            pltpu.SemaphoreType.DMA,
            pltpu.SemaphoreType.DMA,
        ],
        compiler_params=pltpu.CompilerParams(use_tc_tiling_on_sc=False),
    )(gi, si, table)


def _mm_kernel(x_ref, w_ref, o_ref):
    o_ref[...] = jnp.dot(x_ref[...], w_ref[...],
                         preferred_element_type=jnp.float32)


def _matmul(x, w, block_rows):
    n, k = x.shape
    _, m = w.shape
    grid = n // block_rows
    return pl.pallas_call(
        _mm_kernel,
        grid=(grid,),
        in_specs=[
            pl.BlockSpec((block_rows, k), lambda i: (i, 0)),
            pl.BlockSpec((k, m), lambda i: (0, 0)),
        ],
        out_specs=pl.BlockSpec((block_rows, m), lambda i: (i, 0)),
        out_shape=jax.ShapeDtypeStruct((n, m), jnp.float32),
    )(x, w)


def _segsum(vals, seg, num_segments):
    return jax.ops.segment_sum(vals, seg, num_segments=num_segments)


def kernel(x, edge_index, W1, att1, b1, W2, b2, interpret=False):
    idx_node = edge_index[0]
    idx_edge = edge_index[1]
    xw = _matmul(x, W1, 1000)
    xw3 = xw.reshape(_N, _H, _F)
    a_src = (xw3 * att1[:, :, :_F]).sum(-1)
    a_dst = (xw3 * att1[:, :, _F:]).sum(-1)
    c = jax.nn.leaky_relu(a_src.max(0) + a_dst.max(0), negative_slope=0.2)
    c16 = jnp.concatenate([c, jnp.zeros((8,), jnp.float32)])
    t1 = jnp.concatenate([a_src, a_dst], axis=1)
    t2 = jnp.concatenate([a_dst, a_src], axis=1)

    e2, pa, pb = _p1(idx_node, idx_edge, t1, t2, c16, interpret=interpret)
    A = pa[0, :_N] + pa[1, :_N]
    B = pb[0, :_N] + pb[1, :_N]
    s = A[:, :_H]
    deg = A[:, _H]
    bsum = B[:, _H]
    sinv = 1.0 / (s + 1e-16)
    Dinv = jnp.where(deg > 0, 1.0 / deg, 0.0)
    Binv = jnp.where(bsum > 0, 1.0 / bsum, 0.0)

    xs = (xw3 * sinv[:, :, None]).reshape(_N, _HF)
    efp = _p23(idx_node, idx_edge, xs, e2)
    ef = (efp[0, :_N] + efp[1, :_N]) * Binv[:, None]

    hp = _p23(idx_edge, idx_node, ef, e2)
    h = (hp[0, :_N] + hp[1, :_N]).reshape(_N, _H, _F)
    h = h * (sinv * Dinv[:, None])[:, :, None]
    h = jax.nn.elu(h.reshape(_N, _HF) + b1)

    hw = _matmul(h, W2, 1000)
    pad8 = jnp.zeros((_N, 8), jnp.float32)
    hw16 = jnp.concatenate([hw, pad8], axis=1)

    e2p = _p45(idx_node, idx_edge, hw16)
    ef2 = (e2p[0, :_N, :_O2] + e2p[1, :_N, :_O2]) * Binv[:, None]
    ef2_16 = jnp.concatenate([ef2, pad8], axis=1)

    op = _p45(idx_edge, idx_node, ef2_16)
    o = (op[0, :_N, :_O2] + op[1, :_N, :_O2]) * Dinv[:, None]
    o = o + b2
    return jax.nn.log_softmax(o, axis=1)
